# baseline (speedup 1.0000x reference)
import functools

import jax
import jax.numpy as jnp
from jax import lax
from jax.experimental import pallas as pl
from jax.experimental.pallas import tpu as pltpu
from jax.experimental.pallas import tpu_sc as plsc

NC = 2
NS = 16
NW = NC * NS
RB = 8000


def _tc_row_sums(embeddings):
    V, D = embeddings.shape
    QB = RB // 64
    assert D == 64 and V % RB == 0

    def body(x_ref, o_ref):
        x = x_ref[...].astype(jnp.bfloat16)
        ones = jnp.ones((D, 64), jnp.bfloat16)
        o1 = jnp.dot(x, ones, preferred_element_type=jnp.float32)
        o_ref[...] = o1[:, :1].reshape(1, QB, 64)

    return pl.pallas_call(
        body,
        grid=(V // RB,),
        in_specs=[pl.BlockSpec((RB, D), lambda i: (i, 0))],
        out_specs=pl.BlockSpec((1, QB, 64), lambda i: (i, 0, 0)),
        out_shape=jax.ShapeDtypeStruct((V // RB, QB, 64), jnp.float32),
    )(embeddings)


def _sc_gather(pos_words, neg_words, rs_flat):
    B = pos_words.shape[0]
    bpw = B // NW
    nch = bpw // 128

    mesh = plsc.VectorSubcoreMesh(core_axis_name="c", subcore_axis_name="s")

    @functools.partial(
        pl.kernel,
        mesh=mesh,
        compiler_params=pltpu.CompilerParams(needs_layout_passes=False,
                                             use_tc_tiling_on_sc=False),
        out_type=[
            jax.ShapeDtypeStruct((B,), jnp.float32),
            jax.ShapeDtypeStruct((B,), jnp.float32),
        ],
        scratch_types=[
            pltpu.VMEM((nch, 128), jnp.int32),
            pltpu.VMEM((nch, 128), jnp.int32),
            pltpu.VMEM((nch, 128), jnp.float32),
            pltpu.VMEM((nch, 128), jnp.float32),
            pltpu.SemaphoreType.DMA,
        ],
    )
    def sc_kernel(pos_hbm, neg_hbm, rs_hbm, pos_out, neg_out,
                  pidx, nidx, pval, nval, sem):
        wid = lax.axis_index("s") * NC + lax.axis_index("c")
        base = wid * bpw

        for j in range(nch):
            pltpu.sync_copy(pos_hbm.at[pl.ds(base + j * 128, 128)],
                            pidx.at[j])
            pltpu.sync_copy(neg_hbm.at[pl.ds(base + j * 128, 128)],
                            nidx.at[j])
        copies = [
            pltpu.make_async_copy(rs_hbm.at[pidx.at[j]], pval.at[j], sem)
            for j in range(nch)
        ] + [
            pltpu.make_async_copy(rs_hbm.at[nidx.at[j]], nval.at[j], sem)
            for j in range(nch)
        ]
        for c in copies:
            c.start()
        for c in copies:
            c.wait()
        for j in range(nch):
            pltpu.sync_copy(pval.at[j],
                            pos_out.at[pl.ds(base + j * 128, 128)])
            pltpu.sync_copy(nval.at[j],
                            neg_out.at[pl.ds(base + j * 128, 128)])

    return sc_kernel(pos_words, neg_words, rs_flat)


def _finish(pos_sums, neg_sums, batch):
    def body(p_ref, n_ref, o_ref):
        p = p_ref[...]
        n = n_ref[...]
        t = jnp.maximum(-p, 0.0) + jnp.log(1.0 + jnp.exp(-jnp.abs(p)))
        t = t + jnp.maximum(n, 0.0) + jnp.log(1.0 + jnp.exp(-jnp.abs(n)))
        o_ref[0, 0] = jnp.sum(t) * (1.0 / batch)

    out = pl.pallas_call(
        body,
        out_shape=jax.ShapeDtypeStruct((1, 1), jnp.float32),
        out_specs=pl.BlockSpec(memory_space=pltpu.SMEM),
    )(pos_sums, neg_sums)
    return out[0, 0]


def kernel(pos_words, neg_words, embeddings):
    B = pos_words.shape[0]
    V, D = embeddings.shape
    rs = _tc_row_sums(embeddings)
    rs_flat = rs.reshape(V)
    pos_sums, neg_sums = _sc_gather(pos_words.astype(jnp.int32),
                                    neg_words.astype(jnp.int32),
                                    rs_flat)
    return _finish(pos_sums.reshape(128, -1), neg_sums.reshape(128, -1), B)


# speedup vs baseline: 1.1120x; 1.1120x over previous
import functools

import jax
import jax.numpy as jnp
from jax import lax
from jax.experimental import pallas as pl
from jax.experimental.pallas import tpu as pltpu
from jax.experimental.pallas import tpu_sc as plsc

NC = 2
NS = 16
NW = NC * NS
RB = 40000


def _tc_row_sums(embeddings):
    V, D = embeddings.shape
    QB = RB // 64
    assert D == 64 and V % RB == 0

    def body(x_ref, o_ref):
        x = x_ref[...].astype(jnp.bfloat16)
        ones = jnp.ones((D, 64), jnp.bfloat16)
        o1 = jnp.dot(x, ones, preferred_element_type=jnp.float32)
        o_ref[...] = o1[:, :1].reshape(1, QB, 64)

    return pl.pallas_call(
        body,
        grid=(V // RB,),
        in_specs=[pl.BlockSpec((RB, D), lambda i: (i, 0))],
        out_specs=pl.BlockSpec((1, QB, 64), lambda i: (i, 0, 0)),
        out_shape=jax.ShapeDtypeStruct((V // RB, QB, 64), jnp.float32),
    )(embeddings)


def _sc_gather(pos_words, neg_words, rs_flat):
    B = pos_words.shape[0]
    bpw = B // NW
    nch = bpw // 128

    mesh = plsc.VectorSubcoreMesh(core_axis_name="c", subcore_axis_name="s")

    @functools.partial(
        pl.kernel,
        mesh=mesh,
        compiler_params=pltpu.CompilerParams(needs_layout_passes=False,
                                             use_tc_tiling_on_sc=False),
        out_type=[
            jax.ShapeDtypeStruct((B,), jnp.float32),
            jax.ShapeDtypeStruct((B,), jnp.float32),
        ],
        scratch_types=[
            pltpu.VMEM((nch, 128), jnp.int32),
            pltpu.VMEM((nch, 128), jnp.int32),
            pltpu.VMEM((nch, 128), jnp.float32),
            pltpu.VMEM((nch, 128), jnp.float32),
            pltpu.SemaphoreType.DMA,
        ],
    )
    def sc_kernel(pos_hbm, neg_hbm, rs_hbm, pos_out, neg_out,
                  pidx, nidx, pval, nval, sem):
        wid = lax.axis_index("s") * NC + lax.axis_index("c")
        base = wid * bpw

        for j in range(nch):
            pltpu.sync_copy(pos_hbm.at[pl.ds(base + j * 128, 128)],
                            pidx.at[j])
            pltpu.sync_copy(neg_hbm.at[pl.ds(base + j * 128, 128)],
                            nidx.at[j])
        copies = [
            pltpu.make_async_copy(rs_hbm.at[pidx.at[j]], pval.at[j], sem)
            for j in range(nch)
        ] + [
            pltpu.make_async_copy(rs_hbm.at[nidx.at[j]], nval.at[j], sem)
            for j in range(nch)
        ]
        for c in copies:
            c.start()
        for c in copies:
            c.wait()
        for j in range(nch):
            pltpu.sync_copy(pval.at[j],
                            pos_out.at[pl.ds(base + j * 128, 128)])
            pltpu.sync_copy(nval.at[j],
                            neg_out.at[pl.ds(base + j * 128, 128)])

    return sc_kernel(pos_words, neg_words, rs_flat)


def _finish(pos_sums, neg_sums, batch):
    def body(p_ref, n_ref, o_ref):
        p = p_ref[...]
        n = n_ref[...]
        t = jnp.maximum(-p, 0.0) + jnp.log(1.0 + jnp.exp(-jnp.abs(p)))
        t = t + jnp.maximum(n, 0.0) + jnp.log(1.0 + jnp.exp(-jnp.abs(n)))
        o_ref[0, 0] = jnp.sum(t) * (1.0 / batch)

    out = pl.pallas_call(
        body,
        out_shape=jax.ShapeDtypeStruct((1, 1), jnp.float32),
        out_specs=pl.BlockSpec(memory_space=pltpu.SMEM),
    )(pos_sums, neg_sums)
    return out[0, 0]


def kernel(pos_words, neg_words, embeddings):
    B = pos_words.shape[0]
    V, D = embeddings.shape
    rs = _tc_row_sums(embeddings)
    rs_flat = rs.reshape(V)
    pos_sums, neg_sums = _sc_gather(pos_words.astype(jnp.int32),
                                    neg_words.astype(jnp.int32),
                                    rs_flat)
    return _finish(pos_sums.reshape(128, -1), neg_sums.reshape(128, -1), B)


# speedup vs baseline: 1.1135x; 1.0013x over previous
import functools

import jax
import jax.numpy as jnp
from jax import lax
from jax.experimental import pallas as pl
from jax.experimental.pallas import tpu as pltpu
from jax.experimental.pallas import tpu_sc as plsc

NC = 2
NS = 16
NW = NC * NS
RB = 40000


def _tc_row_sums(embeddings):
    V, D = embeddings.shape
    QB = RB // 64
    NSTR = 4
    SB = RB // NSTR
    assert D == 64 and V % RB == 0

    def body(x0, x1, x2, x3, o_ref):
        ones = jnp.ones((D, 64), jnp.bfloat16)
        cols = []
        for xr in (x0, x1, x2, x3):
            x = xr[...].astype(jnp.bfloat16)
            o1 = jnp.dot(x, ones, preferred_element_type=jnp.float32)
            cols.append(o1[:, :1])
        o_ref[...] = jnp.concatenate(cols, axis=0).reshape(1, QB, 64)

    def mk_map(k):
        return lambda i: (i * NSTR + k, 0)

    return pl.pallas_call(
        body,
        grid=(V // RB,),
        in_specs=[pl.BlockSpec((SB, D), mk_map(k)) for k in range(NSTR)],
        out_specs=pl.BlockSpec((1, QB, 64), lambda i: (i, 0, 0)),
        out_shape=jax.ShapeDtypeStruct((V // RB, QB, 64), jnp.float32),
    )(embeddings, embeddings, embeddings, embeddings)


def _sc_gather(pos_words, neg_words, rs_flat):
    B = pos_words.shape[0]
    bpw = B // NW
    nch = bpw // 128

    mesh = plsc.VectorSubcoreMesh(core_axis_name="c", subcore_axis_name="s")

    @functools.partial(
        pl.kernel,
        mesh=mesh,
        compiler_params=pltpu.CompilerParams(needs_layout_passes=False,
                                             use_tc_tiling_on_sc=False),
        out_type=[
            jax.ShapeDtypeStruct((B,), jnp.float32),
            jax.ShapeDtypeStruct((B,), jnp.float32),
        ],
        scratch_types=[
            pltpu.VMEM((nch, 128), jnp.int32),
            pltpu.VMEM((nch, 128), jnp.int32),
            pltpu.VMEM((nch, 128), jnp.float32),
            pltpu.VMEM((nch, 128), jnp.float32),
            pltpu.SemaphoreType.DMA,
        ],
    )
    def sc_kernel(pos_hbm, neg_hbm, rs_hbm, pos_out, neg_out,
                  pidx, nidx, pval, nval, sem):
        wid = lax.axis_index("s") * NC + lax.axis_index("c")
        base = wid * bpw

        for j in range(nch):
            pltpu.sync_copy(pos_hbm.at[pl.ds(base + j * 128, 128)],
                            pidx.at[j])
            pltpu.sync_copy(neg_hbm.at[pl.ds(base + j * 128, 128)],
                            nidx.at[j])
        copies = [
            pltpu.make_async_copy(rs_hbm.at[pidx.at[j]], pval.at[j], sem)
            for j in range(nch)
        ] + [
            pltpu.make_async_copy(rs_hbm.at[nidx.at[j]], nval.at[j], sem)
            for j in range(nch)
        ]
        for c in copies:
            c.start()
        for c in copies:
            c.wait()
        for j in range(nch):
            pltpu.sync_copy(pval.at[j],
                            pos_out.at[pl.ds(base + j * 128, 128)])
            pltpu.sync_copy(nval.at[j],
                            neg_out.at[pl.ds(base + j * 128, 128)])

    return sc_kernel(pos_words, neg_words, rs_flat)


def _finish(pos_sums, neg_sums, batch):
    def body(p_ref, n_ref, o_ref):
        p = p_ref[...]
        n = n_ref[...]
        t = jnp.maximum(-p, 0.0) + jnp.log(1.0 + jnp.exp(-jnp.abs(p)))
        t = t + jnp.maximum(n, 0.0) + jnp.log(1.0 + jnp.exp(-jnp.abs(n)))
        o_ref[0, 0] = jnp.sum(t) * (1.0 / batch)

    out = pl.pallas_call(
        body,
        out_shape=jax.ShapeDtypeStruct((1, 1), jnp.float32),
        out_specs=pl.BlockSpec(memory_space=pltpu.SMEM),
    )(pos_sums, neg_sums)
    return out[0, 0]


def kernel(pos_words, neg_words, embeddings):
    B = pos_words.shape[0]
    V, D = embeddings.shape
    rs = _tc_row_sums(embeddings)
    rs_flat = rs.reshape(V)
    pos_sums, neg_sums = _sc_gather(pos_words.astype(jnp.int32),
                                    neg_words.astype(jnp.int32),
                                    rs_flat)
    return _finish(pos_sums.reshape(128, -1), neg_sums.reshape(128, -1), B)


# speedup vs baseline: 2.3509x; 2.1113x over previous
import functools

import jax
import jax.numpy as jnp
from jax import lax
from jax.experimental import pallas as pl
from jax.experimental.pallas import tpu as pltpu
from jax.experimental.pallas import tpu_sc as plsc

NC = 2
NS = 16
NW = NC * NS
CH = 32


def _sc_row_sums(pos_words, neg_words, emb3):
    B = pos_words.shape[0]
    NT, _, D = emb3.shape
    assert D == 64
    bpw = B // NW
    nchunk = 2 * bpw // CH

    mesh = plsc.VectorSubcoreMesh(core_axis_name="c", subcore_axis_name="s")

    @functools.partial(
        pl.kernel,
        mesh=mesh,
        compiler_params=pltpu.CompilerParams(needs_layout_passes=False,
                                             use_tc_tiling_on_sc=True),
        out_type=[
            jax.ShapeDtypeStruct((B,), jnp.float32),
            jax.ShapeDtypeStruct((B,), jnp.float32),
        ],
        scratch_types=[
            pltpu.VMEM((2 * bpw,), jnp.int32),
            pltpu.VMEM((CH, 8, D), jnp.float32),
            pltpu.VMEM((CH, 8, D), jnp.float32),
            pltpu.VMEM((2 * bpw,), jnp.float32),
            pltpu.SemaphoreType.DMA((4,)),
            pltpu.SemaphoreType.DMA((4,)),
        ],
    )
    def sc_kernel(pos_hbm, neg_hbm, emb_hbm, pos_out, neg_out,
                  idx_s, buf_a, buf_b, sums, sem_a, sem_b):
        wid = lax.axis_index("s") * NC + lax.axis_index("c")
        base = wid * bpw
        lid = lax.iota(jnp.int32, 16)

        for half, idx_hbm in ((0, pos_hbm), (1, neg_hbm)):
            pltpu.sync_copy(idx_hbm.at[pl.ds(base, bpw)],
                            idx_s.at[pl.ds(half * bpw, bpw)])

        def fire(k, buf, sem):
            for g in range(CH // 16):
                wv = idx_s[pl.ds(k * CH + g * 16, 16)]
                tv = wv >> 3
                sv = wv & 7
                for u in range(16):
                    pltpu.make_async_copy(
                        emb_hbm.at[tv[u], sv[u]],
                        buf.at[g * 16 + u, sv[u]], sem.at[u % 4]).start()

        def drain(buf, sem):
            for u in range(CH):
                pltpu.make_async_copy(
                    emb_hbm.at[0, 0], buf.at[u, 0], sem.at[u % 4]).wait()

        def reduce_chunk(k, buf):
            for g in range(CH // 16):
                sv = idx_s[pl.ds(k * CH + g * 16, 16)] & 7
                acc = jnp.zeros((16,), jnp.float32)
                for u in range(16):
                    j = g * 16 + u
                    s = sv[u]
                    v = (buf[j, s, pl.ds(0, 16)] + buf[j, s, pl.ds(16, 16)]
                         + buf[j, s, pl.ds(32, 16)] + buf[j, s, pl.ds(48, 16)])
                    acc = jnp.where(lid == u, jnp.sum(v), acc)
                sums[pl.ds(k * CH + g * 16, 16)] = acc

        fire(0, buf_a, sem_a)
        fire(1, buf_b, sem_b)

        def body(i, _):
            k = i * 2
            drain(buf_a, sem_a)

            @pl.when(k + 2 < nchunk)
            def _():
                fire(k + 2, buf_a, sem_a)
            reduce_chunk(k, buf_a)

            drain(buf_b, sem_b)

            @pl.when(k + 3 < nchunk)
            def _():
                fire(k + 3, buf_b, sem_b)
            reduce_chunk(k + 1, buf_b)
            return _
        lax.fori_loop(0, nchunk // 2, body, None)

        pltpu.sync_copy(sums.at[pl.ds(0, bpw)], pos_out.at[pl.ds(base, bpw)])
        pltpu.sync_copy(sums.at[pl.ds(bpw, bpw)], neg_out.at[pl.ds(base, bpw)])

    return sc_kernel(pos_words, neg_words, emb3)


def _finish(pos_sums, neg_sums, batch):
    def body(p_ref, n_ref, o_ref):
        p = p_ref[...]
        n = n_ref[...]
        t = jnp.maximum(-p, 0.0) + jnp.log(1.0 + jnp.exp(-jnp.abs(p)))
        t = t + jnp.maximum(n, 0.0) + jnp.log(1.0 + jnp.exp(-jnp.abs(n)))
        o_ref[0, 0] = jnp.sum(t) * (1.0 / batch)

    out = pl.pallas_call(
        body,
        out_shape=jax.ShapeDtypeStruct((1, 1), jnp.float32),
        out_specs=pl.BlockSpec(memory_space=pltpu.SMEM),
    )(pos_sums, neg_sums)
    return out[0, 0]


def kernel(pos_words, neg_words, embeddings):
    B = pos_words.shape[0]
    V, D = embeddings.shape
    emb3 = embeddings.reshape(V // 8, 8, D)
    pos_sums, neg_sums = _sc_row_sums(pos_words.astype(jnp.int32),
                                      neg_words.astype(jnp.int32),
                                      emb3)
    return _finish(pos_sums.reshape(128, -1), neg_sums.reshape(128, -1), B)
